# Initial kernel scaffold; baseline (speedup 1.0000x reference)
import jax
import jax.numpy as jnp
from jax.experimental import pallas as pl


def kernel(x, edge_index, w, W, b):
    raise NotImplementedError("write your pallas kernel here")


# speedup vs baseline: 6.4801x; 6.4801x over previous
import functools

import jax
import jax.numpy as jnp
from jax import lax
from jax.experimental import pallas as pl
from jax.experimental.pallas import tpu as pltpu
from jax.experimental.pallas import tpu_sc as plsc

N_NODES = 10000
N_EDGES = 320000
D_FEAT = 128
DP = 144
NC = 2
NS = 16
NW = NC * NS
CH = 128
EROWS = N_EDGES // CH
BASE_ROWS = EROWS // NW
EXTRA = EROWS - BASE_ROWS * NW
NP = 10240
ROWS_PER_TILE = NP // NS
OCP = 128


def _sc_aggregate(xp, src, dst):
    mesh = plsc.VectorSubcoreMesh(
        core_axis_name="c", subcore_axis_name="s",
        num_cores=NC, num_subcores=NS)

    @functools.partial(
        pl.kernel,
        out_type=jax.ShapeDtypeStruct((NC, NP, DP), jnp.float32),
        mesh=mesh,
        scratch_types=[
            pltpu.VMEM((CH,), jnp.int32),
            pltpu.VMEM((CH,), jnp.int32),
            pltpu.VMEM((CH, DP), jnp.float32),
            pltpu.VMEM_SHARED((NP, DP), jnp.float32),
            pltpu.SemaphoreType.DMA,
        ],
        compiler_params=pltpu.CompilerParams(use_tc_tiling_on_sc=False),
    )
    def agg(xp_hbm, src_hbm, dst_hbm, out_hbm, idx_s, idx_d, rows_v, acc, sem):
        c = lax.axis_index("c")
        s = lax.axis_index("s")
        wid = c * NS + s

        zeros16 = jnp.zeros((16,), jnp.float32)

        def zb(i, carry):
            r = i // (DP // 16)
            j = i % (DP // 16)
            rows_v[r, pl.ds(j * 16, 16)] = zeros16
            return carry

        lax.fori_loop(0, CH * (DP // 16), zb, 0)
        tb = s * ROWS_PER_TILE
        for kk in range(ROWS_PER_TILE // OCP):
            pltpu.sync_copy(rows_v, acc.at[pl.ds(tb + kk * OCP, OCP)])
        plsc.subcore_barrier()

        base_row = wid * BASE_ROWS + jnp.minimum(wid, EXTRA)
        nrows = BASE_ROWS + jnp.where(wid < EXTRA, 1, 0)

        def eb(r, carry):
            ebase = (base_row + r) * CH
            pltpu.sync_copy(src_hbm.at[pl.ds(ebase, CH)], idx_s)
            pltpu.sync_copy(dst_hbm.at[pl.ds(ebase, CH)], idx_d)
            pltpu.async_copy(xp_hbm.at[idx_s], rows_v, sem).wait()
            pltpu.sync_copy(rows_v, acc.at[idx_d], add=True)
            return carry

        lax.fori_loop(0, nrows, eb, 0)
        plsc.subcore_barrier()

        for kk in range(ROWS_PER_TILE // OCP):
            rb = tb + kk * OCP
            pltpu.sync_copy(acc.at[pl.ds(rb, OCP)], rows_v)
            pltpu.sync_copy(rows_v, out_hbm.at[c, pl.ds(rb, OCP)])

    return agg(xp, src, dst)


BLK = 400


def _tc_finish(x, parts, Wt, b2):
    def body(x_ref, p_ref, wt_ref, b_ref, o_ref):
        xb = x_ref[...]
        ps = p_ref[0] + p_ref[1]
        msum = ps[:, :D_FEAT]
        cnt = ps[:, D_FEAT:D_FEAT + 1]
        y = jnp.where(cnt > 0, msum / jnp.maximum(cnt, 1.0), 0.0)
        h = jnp.concatenate([xb, y], axis=1)
        o_ref[...] = (jnp.dot(h, wt_ref[...],
                              preferred_element_type=jnp.float32)
                      + b_ref[...])

    return pl.pallas_call(
        body,
        grid=(N_NODES // BLK,),
        in_specs=[
            pl.BlockSpec((BLK, D_FEAT), lambda i: (i, 0)),
            pl.BlockSpec((NC, BLK, DP), lambda i: (0, i, 0)),
            pl.BlockSpec((2 * D_FEAT, D_FEAT), lambda i: (0, 0)),
            pl.BlockSpec((1, D_FEAT), lambda i: (0, 0)),
        ],
        out_specs=pl.BlockSpec((BLK, D_FEAT), lambda i: (i, 0)),
        out_shape=jax.ShapeDtypeStruct((N_NODES, D_FEAT), jnp.float32),
    )(x, parts, Wt, b2)


def kernel(x, edge_index, w, W, b):
    src = edge_index[0]
    dst = edge_index[1]
    xp = jnp.concatenate(
        [x, jnp.ones((N_NODES, 1), jnp.float32),
         jnp.zeros((N_NODES, DP - D_FEAT - 1), jnp.float32)], axis=1)
    parts = _sc_aggregate(xp, src, dst)
    return _tc_finish(x, parts, W.T, b.reshape(1, D_FEAT))


# speedup vs baseline: 9.0308x; 1.3936x over previous
import functools

import jax
import jax.numpy as jnp
from jax import lax
from jax.experimental import pallas as pl
from jax.experimental.pallas import tpu as pltpu
from jax.experimental.pallas import tpu_sc as plsc

N_NODES = 10000
N_EDGES = 320000
D_FEAT = 128
DP = 144
NC = 2
NS = 16
NW = NC * NS
CH = 128
CPW = 80
EP = NW * CPW * CH
NB = 2
NP = 10240
ROWS_PER_TILE = NP // NS
OCP = 128


def _sc_aggregate(xp, src2d, dst2d):
    mesh = plsc.VectorSubcoreMesh(
        core_axis_name="c", subcore_axis_name="s",
        num_cores=NC, num_subcores=NS)

    @functools.partial(
        pl.kernel,
        out_type=jax.ShapeDtypeStruct((NC, NP, DP), jnp.float32),
        mesh=mesh,
        scratch_types=[
            pltpu.VMEM((NB, CH), jnp.int32),
            pltpu.VMEM((NB, CH), jnp.int32),
            pltpu.VMEM((NB, CH, DP), jnp.float32),
            pltpu.VMEM_SHARED((NP, DP), jnp.float32),
            pltpu.SemaphoreType.DMA,
        ],
        compiler_params=pltpu.CompilerParams(use_tc_tiling_on_sc=False),
    )
    def agg(xp_hbm, src_hbm, dst_hbm, out_hbm, idx_s, idx_d, rows, acc, gsem):
        c = lax.axis_index("c")
        s = lax.axis_index("s")
        wid = c * NS + s

        zeros16 = jnp.zeros((16,), jnp.float32)

        def zb(i, carry):
            r = i // (DP // 16)
            j = i % (DP // 16)
            rows[0, r, pl.ds(j * 16, 16)] = zeros16
            return carry

        lax.fori_loop(0, CH * (DP // 16), zb, 0)
        tb = s * ROWS_PER_TILE
        for kk in range(ROWS_PER_TILE // OCP):
            pltpu.sync_copy(rows.at[0], acc.at[pl.ds(tb + kk * OCP, OCP)])
        plsc.subcore_barrier()

        wbase_e = wid * CPW * CH

        def stage(chunk, b):
            pltpu.sync_copy(src_hbm.at[pl.ds(wbase_e + chunk * CH, CH)],
                            idx_s.at[b])
            pltpu.sync_copy(dst_hbm.at[pl.ds(wbase_e + chunk * CH, CH)],
                            idx_d.at[b])

        def fire(b):
            pltpu.async_copy(xp_hbm.at[idx_s.at[b]], rows.at[b], gsem)

        def drain(b):
            pltpu.make_async_copy(
                xp_hbm.at[pl.ds(0, CH)], rows.at[b], gsem).wait()

        def scat(b):
            pltpu.sync_copy(rows.at[b], acc.at[idx_d.at[b]], add=True)

        for b in range(NB):
            stage(b, b)
            fire(b)

        def outer(o, carry):
            for b in range(NB):
                chunk = o * NB + b
                drain(b)
                scat(b)
                nxt = chunk + NB

                @pl.when(nxt < CPW)
                def _():
                    stage(nxt, b)
                    fire(b)
            return carry

        lax.fori_loop(0, CPW // NB, outer, 0)
        plsc.subcore_barrier()

        for kk in range(ROWS_PER_TILE // OCP):
            rb = tb + kk * OCP
            pltpu.sync_copy(acc.at[pl.ds(rb, OCP)], rows.at[0])
            pltpu.sync_copy(rows.at[0], out_hbm.at[c, pl.ds(rb, OCP)])

    return agg(xp, src2d, dst2d)


BLK = 400


def _tc_finish(x, parts, Wt, b2):
    def body(x_ref, p_ref, wt_ref, b_ref, o_ref):
        xb = x_ref[...]
        ps = p_ref[0] + p_ref[1]
        msum = ps[:, :D_FEAT]
        cnt = ps[:, D_FEAT:D_FEAT + 1]
        y = jnp.where(cnt > 0, msum / jnp.maximum(cnt, 1.0), 0.0)
        h = jnp.concatenate([xb, y], axis=1)
        o_ref[...] = (jnp.dot(h, wt_ref[...],
                              preferred_element_type=jnp.float32)
                      + b_ref[...])

    return pl.pallas_call(
        body,
        grid=(N_NODES // BLK,),
        in_specs=[
            pl.BlockSpec((BLK, D_FEAT), lambda i: (i, 0)),
            pl.BlockSpec((NC, BLK, DP), lambda i: (0, i, 0)),
            pl.BlockSpec((2 * D_FEAT, D_FEAT), lambda i: (0, 0)),
            pl.BlockSpec((1, D_FEAT), lambda i: (0, 0)),
        ],
        out_specs=pl.BlockSpec((BLK, D_FEAT), lambda i: (i, 0)),
        out_shape=jax.ShapeDtypeStruct((N_NODES, D_FEAT), jnp.float32),
    )(x, parts, Wt, b2)


def kernel(x, edge_index, w, W, b):
    src = edge_index[0]
    dst = edge_index[1]
    npad = EP - N_EDGES
    pad_i = jnp.arange(npad, dtype=jnp.int32)
    src_p = jnp.concatenate([src, pad_i % N_NODES])
    dst_p = jnp.concatenate([dst, N_NODES + pad_i % (NP - N_NODES)])
    xp = jnp.concatenate(
        [x, jnp.ones((N_NODES, 1), jnp.float32),
         jnp.zeros((N_NODES, DP - D_FEAT - 1), jnp.float32)], axis=1)
    parts = _sc_aggregate(xp, src_p, dst_p)
    return _tc_finish(x, parts, W.T, b.reshape(1, D_FEAT))
